# baseline (speedup 1.0000x reference)
import functools

import jax
import jax.numpy as jnp
from jax import lax
from jax.experimental import pallas as pl
from jax.experimental.pallas import tpu as pltpu
from jax.experimental.pallas import tpu_sc as plsc

B, L, D = 4096, 20, 128
N = B * L
NF = 6
NG = 4
VPAD = 1024
NW = 32
CH = 64
PER_W = N // NW
NIT = PER_W // CH
NSUP = NIT // 2

R_TV, R_BAR, R_PD, R_WORD = 0, 4096, 4096 + 256, 4096 + 256 + 16384
TAB_ROWS = R_WORD + 1024
TV_BLKS, BAR_BLKS, PD_BLKS, WORD_BLKS = 32, 2, 128, 8


def _build_body(w0, w1, w2, w3, w4, w5, p_ref, b_ref, o_ref):
    def fuse(w, f):
        return lax.dot_general(
            w[...], p_ref[:, pl.ds(f * D, D)], (((1,), (1,)), ((), ())),
            preferred_element_type=jnp.float32,
        )

    tempo, bar, pos = fuse(w0, 0), fuse(w1, 1), fuse(w2, 2)
    word, vel, dur = fuse(w3, 3), fuse(w4, 4), fuse(w5, 5)
    tv = tempo[:, None, :] + (vel + b_ref[...])[None, :, :]
    o_ref[pl.ds(R_TV, R_BAR)] = tv.reshape(R_BAR, D)
    o_ref[pl.ds(R_BAR, 256)] = bar
    pd = pos[:, None, :] + dur[None, :, :]
    o_ref[pl.ds(R_PD, R_WORD - R_PD)] = pd.reshape(R_WORD - R_PD, D)
    o_ref[pl.ds(R_WORD, 1024)] = word


_build = pl.pallas_call(
    _build_body,
    out_shape=jax.ShapeDtypeStruct((TAB_ROWS, D), jnp.float32),
)


def _sc_body(i0, i1, i2, i3, i4, i5, tab_hbm, out_hbm,
             idxv, gv0, gv1, ov0, ov1, semg, semo):
    wid = lax.axis_index("s") * 2 + lax.axis_index("c")
    base = wid * PER_W

    cps = [
        pltpu.async_copy(idx.at[pl.ds(base, PER_W)],
                         idxv.at[pl.ds(f * PER_W, PER_W)], semg)
        for f, idx in enumerate((i0, i1, i2, i3, i4, i5))
    ]
    for cp in cps:
        cp.wait()

    def comb_body(p, carry):
        s0 = pl.ds(0 * PER_W + p * 16, 16)
        s1 = pl.ds(1 * PER_W + p * 16, 16)
        s2 = pl.ds(2 * PER_W + p * 16, 16)
        s3 = pl.ds(3 * PER_W + p * 16, 16)
        s4 = pl.ds(4 * PER_W + p * 16, 16)
        s5 = pl.ds(5 * PER_W + p * 16, 16)
        idxv[s0] = idxv[s0] * 64 + idxv[s4]
        idxv[s1] = idxv[s1] + jnp.int32(R_BAR)
        idxv[s2] = idxv[s2] * 128 + idxv[s5] + jnp.int32(R_PD)
        idxv[s3] = idxv[s3] + jnp.int32(R_WORD)
        return carry

    lax.fori_loop(0, PER_W // 16, comb_body, 0, unroll=4)

    def fire(t, gvb):
        for g in range(NG):
            pltpu.async_copy(
                tab_hbm.at[idxv.at[pl.ds(g * PER_W + t * CH, CH)]],
                gvb.at[pl.ds(g * CH, CH)], semg)

    def drain_gathers(gvb):
        pltpu.make_async_copy(tab_hbm.at[pl.ds(0, NG * CH)], gvb, semg).wait()

    def drain_out(ovb):
        pltpu.make_async_copy(ovb, out_hbm.at[pl.ds(base, CH)], semo).wait()

    def compute(gvb, ovb):
        def tok(j, carry):
            for c in range(8):
                acc = gvb[j, pl.ds(c * 16, 16)]
                for g in range(1, NG):
                    acc = acc + gvb[g * CH + j, pl.ds(c * 16, 16)]
                ovb[j, pl.ds(c * 16, 16)] = acc
            return carry
        lax.fori_loop(0, CH, tok, 0, unroll=2)

    fire(0, gv0)
    fire(1, gv1)

    def sup(s, carry):
        for half, (gvb, ovb) in enumerate(((gv0, ov0), (gv1, ov1))):
            t = 2 * s + half
            drain_gathers(gvb)

            @pl.when(s > 0)
            def _():
                drain_out(ovb)

            compute(gvb, ovb)
            pltpu.async_copy(ovb, out_hbm.at[pl.ds(base + t * CH, CH)], semo)

            @pl.when(s < NSUP - 1)
            def _():
                fire(t + 2, gvb)
        return carry

    lax.fori_loop(0, NSUP, sup, 0)
    drain_out(ov0)
    drain_out(ov1)


_sc_gather = functools.partial(
    pl.kernel,
    out_type=jax.ShapeDtypeStruct((N, D), jnp.float32),
    mesh=plsc.VectorSubcoreMesh(core_axis_name="c", subcore_axis_name="s",
                                num_cores=2),
    scratch_types=[
        pltpu.VMEM((NF * PER_W,), jnp.int32),
        pltpu.VMEM((NG * CH, D), jnp.float32),
        pltpu.VMEM((NG * CH, D), jnp.float32),
        pltpu.VMEM((CH, D), jnp.float32),
        pltpu.VMEM((CH, D), jnp.float32),
        pltpu.SemaphoreType.DMA,
        pltpu.SemaphoreType.DMA,
    ],
)(_sc_body)


@jax.jit
def kernel(tempo, global_bar, global_pos, token, vel, dur,
           W_tempo, W_bar, W_pos, W_word, W_vel, W_dur, proj_W, proj_b):
    tab = _build(W_tempo, W_bar, W_pos, W_word, W_vel, W_dur,
                 proj_W, proj_b.reshape(1, D))

    idxs = [a.astype(jnp.int32).reshape(N)
            for a in (tempo, global_bar, global_pos, token, vel, dur)]
    out = _sc_gather(*idxs, tab)
    return out.reshape(B, L, D)


# speedup vs baseline: 1.2606x; 1.2606x over previous
import functools

import jax
import jax.numpy as jnp
from jax import lax
from jax.experimental import pallas as pl
from jax.experimental.pallas import tpu as pltpu
from jax.experimental.pallas import tpu_sc as plsc

B, L, D = 4096, 20, 128
N = B * L
NF = 6
NG = 4
VPAD = 1024
NW = 32
CH = 40
BPC = CH // L
PER_W = N // NW
PER_WB = PER_W // L
NIT = PER_W // CH
NSUP = NIT // 2

R_TV, R_BAR, R_PD, R_WORD = 0, 4096, 4096 + 256, 4096 + 256 + 16384
TAB_ROWS = R_WORD + 1024
TV_BLKS, BAR_BLKS, PD_BLKS, WORD_BLKS = 32, 2, 128, 8


def _build_body(w0, w1, w2, w3, w4, w5, p_ref, b_ref, o_ref):
    def fuse(w, f):
        return lax.dot_general(
            w[...], p_ref[:, pl.ds(f * D, D)], (((1,), (1,)), ((), ())),
            preferred_element_type=jnp.float32,
        )

    tempo, bar, pos = fuse(w0, 0), fuse(w1, 1), fuse(w2, 2)
    word, vel, dur = fuse(w3, 3), fuse(w4, 4), fuse(w5, 5)
    tv = tempo[:, None, :] + (vel + b_ref[...])[None, :, :]
    o_ref[pl.ds(R_TV, R_BAR)] = tv.reshape(R_BAR, D)
    o_ref[pl.ds(R_BAR, 256)] = bar
    pd = pos[:, None, :] + dur[None, :, :]
    o_ref[pl.ds(R_PD, R_WORD - R_PD)] = pd.reshape(R_WORD - R_PD, D)
    o_ref[pl.ds(R_WORD, 1024)] = word


_build = pl.pallas_call(
    _build_body,
    out_shape=jax.ShapeDtypeStruct((TAB_ROWS, D), jnp.float32),
)


def _sc_body(i0, i1, i2, i3, i4, i5, tab_hbm, out_hbm,
             idxv, gv0, gv1, ov0, ov1, semg, semo):
    wid = lax.axis_index("s") * 2 + lax.axis_index("c")
    base = wid * PER_W
    bbase = wid * PER_WB

    cps = [
        pltpu.async_copy(idx.at[pl.ds(base, PER_W)],
                         idxv.at[pl.ds(f * PER_W, PER_W)], semg)
        for f, idx in enumerate((i0, i1, i2, i3, i4, i5))
    ]
    for cp in cps:
        cp.wait()

    def comb_body(p, carry):
        s0 = pl.ds(0 * PER_W + p * 16, 16)
        s1 = pl.ds(1 * PER_W + p * 16, 16)
        s2 = pl.ds(2 * PER_W + p * 16, 16)
        s3 = pl.ds(3 * PER_W + p * 16, 16)
        s4 = pl.ds(4 * PER_W + p * 16, 16)
        s5 = pl.ds(5 * PER_W + p * 16, 16)
        idxv[s0] = idxv[s0] * 64 + idxv[s4]
        idxv[s1] = idxv[s1] + jnp.int32(R_BAR)
        idxv[s2] = idxv[s2] * 128 + idxv[s5] + jnp.int32(R_PD)
        idxv[s3] = idxv[s3] + jnp.int32(R_WORD)
        return carry

    lax.fori_loop(0, PER_W // 16, comb_body, 0, unroll=4)

    def fire(t, gvb):
        for g in range(NG):
            pltpu.async_copy(
                tab_hbm.at[idxv.at[pl.ds(g * PER_W + t * CH, CH)]],
                gvb.at[pl.ds(g * CH, CH)], semg)

    def drain_gathers(gvb):
        pltpu.make_async_copy(tab_hbm.at[pl.ds(0, NG * CH)], gvb, semg).wait()

    def drain_out(ovb):
        pltpu.make_async_copy(ovb, out_hbm.at[pl.ds(bbase, BPC)], semo).wait()

    def compute(gvb, ovb):
        for bb in range(BPC):
            def tok(l, carry):
                for c in range(8):
                    acc = gvb[bb * L + l, pl.ds(c * 16, 16)]
                    for g in range(1, NG):
                        acc = acc + gvb[g * CH + bb * L + l,
                                        pl.ds(c * 16, 16)]
                    ovb[bb, l, pl.ds(c * 16, 16)] = acc
                return carry
            lax.fori_loop(0, L, tok, 0, unroll=2)

    fire(0, gv0)
    fire(1, gv1)

    def sup(s, carry):
        for half, (gvb, ovb) in enumerate(((gv0, ov0), (gv1, ov1))):
            t = 2 * s + half
            drain_gathers(gvb)

            @pl.when(s > 0)
            def _():
                drain_out(ovb)

            compute(gvb, ovb)
            pltpu.async_copy(
                ovb, out_hbm.at[pl.ds(bbase + t * BPC, BPC)], semo)

            @pl.when(s < NSUP - 1)
            def _():
                fire(t + 2, gvb)
        return carry

    lax.fori_loop(0, NSUP, sup, 0)
    drain_out(ov0)
    drain_out(ov1)


_sc_gather = functools.partial(
    pl.kernel,
    out_type=jax.ShapeDtypeStruct((B, L, D), jnp.float32),
    mesh=plsc.VectorSubcoreMesh(core_axis_name="c", subcore_axis_name="s",
                                num_cores=2),
    scratch_types=[
        pltpu.VMEM((NF * PER_W,), jnp.int32),
        pltpu.VMEM((NG * CH, D), jnp.float32),
        pltpu.VMEM((NG * CH, D), jnp.float32),
        pltpu.VMEM((BPC, L, D), jnp.float32),
        pltpu.VMEM((BPC, L, D), jnp.float32),
        pltpu.SemaphoreType.DMA,
        pltpu.SemaphoreType.DMA,
    ],
)(_sc_body)


@jax.jit
def kernel(tempo, global_bar, global_pos, token, vel, dur,
           W_tempo, W_bar, W_pos, W_word, W_vel, W_dur, proj_W, proj_b):
    tab = _build(W_tempo, W_bar, W_pos, W_word, W_vel, W_dur,
                 proj_W, proj_b.reshape(1, D))

    idxs = [a.astype(jnp.int32).reshape(N)
            for a in (tempo, global_bar, global_pos, token, vel, dur)]
    return _sc_gather(*idxs, tab)


# speedup vs baseline: 1.2991x; 1.0305x over previous
import functools

import jax
import jax.numpy as jnp
from jax import lax
from jax.experimental import pallas as pl
from jax.experimental.pallas import tpu as pltpu
from jax.experimental.pallas import tpu_sc as plsc

B, L, D = 4096, 20, 128
N = B * L
NF = 6
NG = 4
VPAD = 1024
NW = 32
CH = 80
BPC = CH // L
PER_W = N // NW
PER_WB = PER_W // L
NIT = PER_W // CH
NSUP = NIT // 2

R_TV, R_BAR, R_PD, R_WORD = 0, 4096, 4096 + 256, 4096 + 256 + 16384
TAB_ROWS = R_WORD + 1024
TV_BLKS, BAR_BLKS, PD_BLKS, WORD_BLKS = 32, 2, 128, 8


def _build_body(w0, w1, w2, w3, w4, w5, p_ref, b_ref, o_ref):
    def fuse(w, f):
        return lax.dot_general(
            w[...], p_ref[:, pl.ds(f * D, D)], (((1,), (1,)), ((), ())),
            preferred_element_type=jnp.float32,
        )

    tempo, bar, pos = fuse(w0, 0), fuse(w1, 1), fuse(w2, 2)
    word, vel, dur = fuse(w3, 3), fuse(w4, 4), fuse(w5, 5)
    tv = tempo[:, None, :] + (vel + b_ref[...])[None, :, :]
    o_ref[pl.ds(R_TV, R_BAR)] = tv.reshape(R_BAR, D)
    o_ref[pl.ds(R_BAR, 256)] = bar
    pd = pos[:, None, :] + dur[None, :, :]
    o_ref[pl.ds(R_PD, R_WORD - R_PD)] = pd.reshape(R_WORD - R_PD, D)
    o_ref[pl.ds(R_WORD, 1024)] = word


_build = pl.pallas_call(
    _build_body,
    out_shape=jax.ShapeDtypeStruct((TAB_ROWS, D), jnp.float32),
)


def _sc_body(i0, i1, i2, i3, i4, i5, tab_hbm, out_hbm,
             idxv, gv0, gv1, ov0, ov1, semg, semo):
    wid = lax.axis_index("s") * 2 + lax.axis_index("c")
    base = wid * PER_W
    bbase = wid * PER_WB

    cps = [
        pltpu.async_copy(idx.at[pl.ds(base, PER_W)],
                         idxv.at[pl.ds(f * PER_W, PER_W)], semg)
        for f, idx in enumerate((i0, i1, i2, i3, i4, i5))
    ]
    for cp in cps:
        cp.wait()

    def comb_body(p, carry):
        s0 = pl.ds(0 * PER_W + p * 16, 16)
        s1 = pl.ds(1 * PER_W + p * 16, 16)
        s2 = pl.ds(2 * PER_W + p * 16, 16)
        s3 = pl.ds(3 * PER_W + p * 16, 16)
        s4 = pl.ds(4 * PER_W + p * 16, 16)
        s5 = pl.ds(5 * PER_W + p * 16, 16)
        idxv[s0] = idxv[s0] * 64 + idxv[s4]
        idxv[s1] = idxv[s1] + jnp.int32(R_BAR)
        idxv[s2] = idxv[s2] * 128 + idxv[s5] + jnp.int32(R_PD)
        idxv[s3] = idxv[s3] + jnp.int32(R_WORD)
        return carry

    lax.fori_loop(0, PER_W // 16, comb_body, 0, unroll=4)

    def fire(t, gvb):
        for g in range(NG):
            pltpu.async_copy(
                tab_hbm.at[idxv.at[pl.ds(g * PER_W + t * CH, CH)]],
                gvb.at[pl.ds(g * CH, CH)], semg)

    def drain_gathers(gvb):
        pltpu.make_async_copy(tab_hbm.at[pl.ds(0, NG * CH)], gvb, semg).wait()

    def drain_out(ovb):
        pltpu.make_async_copy(ovb, out_hbm.at[pl.ds(bbase, BPC)], semo).wait()

    def compute(gvb, ovb):
        for bb in range(BPC):
            def tok(l, carry):
                for c in range(8):
                    acc = gvb[bb * L + l, pl.ds(c * 16, 16)]
                    for g in range(1, NG):
                        acc = acc + gvb[g * CH + bb * L + l,
                                        pl.ds(c * 16, 16)]
                    ovb[bb, l, pl.ds(c * 16, 16)] = acc
                return carry
            lax.fori_loop(0, L, tok, 0, unroll=2)

    fire(0, gv0)
    fire(1, gv1)

    def sup(s, carry):
        for half, (gvb, ovb) in enumerate(((gv0, ov0), (gv1, ov1))):
            t = 2 * s + half
            drain_gathers(gvb)

            @pl.when(s > 0)
            def _():
                drain_out(ovb)

            compute(gvb, ovb)
            pltpu.async_copy(
                ovb, out_hbm.at[pl.ds(bbase + t * BPC, BPC)], semo)

            @pl.when(s < NSUP - 1)
            def _():
                fire(t + 2, gvb)
        return carry

    lax.fori_loop(0, NSUP, sup, 0)
    drain_out(ov0)
    drain_out(ov1)


_sc_gather = functools.partial(
    pl.kernel,
    out_type=jax.ShapeDtypeStruct((B, L, D), jnp.float32),
    mesh=plsc.VectorSubcoreMesh(core_axis_name="c", subcore_axis_name="s",
                                num_cores=2),
    scratch_types=[
        pltpu.VMEM((NF * PER_W,), jnp.int32),
        pltpu.VMEM((NG * CH, D), jnp.float32),
        pltpu.VMEM((NG * CH, D), jnp.float32),
        pltpu.VMEM((BPC, L, D), jnp.float32),
        pltpu.VMEM((BPC, L, D), jnp.float32),
        pltpu.SemaphoreType.DMA,
        pltpu.SemaphoreType.DMA,
    ],
)(_sc_body)


@jax.jit
def kernel(tempo, global_bar, global_pos, token, vel, dur,
           W_tempo, W_bar, W_pos, W_word, W_vel, W_dur, proj_W, proj_b):
    tab = _build(W_tempo, W_bar, W_pos, W_word, W_vel, W_dur,
                 proj_W, proj_b.reshape(1, D))

    idxs = [a.astype(jnp.int32).reshape(N)
            for a in (tempo, global_bar, global_pos, token, vel, dur)]
    return _sc_gather(*idxs, tab)
